# baseline (speedup 1.0000x reference)
import functools

import jax
import jax.numpy as jnp
from jax import lax
from jax.experimental import pallas as pl
from jax.experimental.pallas import tpu as pltpu
from jax.experimental.pallas import tpu_sc as plsc

N = 10000
E = 320000
NC, NS = 2, 16
NW = NC * NS
CHUNKS = 80
EC = 125
RPT = N // NS
D1 = 16
D2P = 40
RB = 10000
NBUF = 4


def _make_deg_kernel():
    mesh = plsc.VectorSubcoreMesh(core_axis_name="c", subcore_axis_name="s")

    @functools.partial(
        pl.kernel, mesh=mesh,
        out_type=jax.ShapeDtypeStruct((NC, N, 16), jnp.float32),
        compiler_params=pltpu.CompilerParams(use_tc_tiling_on_sc=False),
        scratch_types=[
            pltpu.VMEM((CHUNKS, EC), jnp.int32),
            pltpu.VMEM((EC, 16), jnp.float32),
            pltpu.VMEM_SHARED((N, 16), jnp.float32),
        ],
    )
    def k(er_hbm, out_hbm, dst_v, obuf, acc):
        cid = lax.axis_index("c")
        sid = lax.axis_index("s")
        wid = sid * NC + cid

        def zrow(i, _):
            obuf[i, pl.ds(0, 16)] = jnp.zeros((16,), jnp.float32)
            return 0
        lax.fori_loop(0, EC, zrow, 0)
        r0 = sid * RPT
        for b in range(RPT // EC):
            pltpu.sync_copy(obuf, acc.at[pl.ds(r0 + b * EC, EC), :])

        def orow(i, _):
            obuf[i, pl.ds(0, 16)] = jnp.ones((16,), jnp.float32)
            return 0
        lax.fori_loop(0, EC, orow, 0)
        pltpu.sync_copy(er_hbm.at[1, wid], dst_v)
        plsc.subcore_barrier()

        def body(j, _):
            pltpu.sync_copy(obuf, acc.at[dst_v.at[j]], add=True)
            return 0
        lax.fori_loop(0, CHUNKS, body, 0)

        plsc.subcore_barrier()
        pltpu.sync_copy(acc.at[pl.ds(r0, RPT), :],
                        out_hbm.at[cid, pl.ds(r0, RPT), :])

    return k


def _make_msg_kernel(D):
    mesh = plsc.VectorSubcoreMesh(core_axis_name="c", subcore_axis_name="s")

    @functools.partial(
        pl.kernel, mesh=mesh,
        out_type=jax.ShapeDtypeStruct((NC, N, D), jnp.float32),
        compiler_params=pltpu.CompilerParams(use_tc_tiling_on_sc=False),
        scratch_types=[
            pltpu.VMEM((CHUNKS, EC), jnp.int32),
            pltpu.VMEM((CHUNKS, EC), jnp.int32),
            [pltpu.VMEM((EC, D), jnp.float32) for _ in range(NBUF)],
            [pltpu.SemaphoreType.DMA for _ in range(NBUF)],
            pltpu.VMEM((RPT, D), jnp.float32),
            pltpu.SemaphoreType.DMA,
            pltpu.VMEM_SHARED((N, D), jnp.float32),
            pltpu.VMEM_SHARED((N, D), jnp.float32),
        ],
    )
    def k(table_hbm, er_hbm, zeros_hbm, out_hbm, src_v, dst_v,
          gbufs, sems, stage_v, sem_s, table_sh, acc):
        cid = lax.axis_index("c")
        sid = lax.axis_index("s")
        wid = sid * NC + cid
        r0 = sid * RPT

        pltpu.async_copy(table_hbm.at[pl.ds(r0, RPT), :], stage_v, sem_s)
        pltpu.sync_copy(zeros_hbm, acc.at[pl.ds(r0, RPT), :])
        pltpu.sync_copy(er_hbm.at[0, wid], src_v)
        pltpu.sync_copy(er_hbm.at[1, wid], dst_v)
        pltpu.make_async_copy(
            table_hbm.at[pl.ds(r0, RPT), :], stage_v, sem_s).wait()
        pltpu.sync_copy(stage_v, table_sh.at[pl.ds(r0, RPT), :])
        plsc.subcore_barrier()

        for b in range(NBUF):
            pltpu.async_copy(table_sh.at[src_v.at[b]], gbufs[b], sems[b])

        def group(g, _):
            base = g * NBUF
            for b in range(NBUF):
                j = base + b
                pltpu.make_async_copy(
                    table_sh.at[src_v.at[j]], gbufs[b], sems[b]).wait()
                pltpu.sync_copy(gbufs[b], acc.at[dst_v.at[j]], add=True)
                jn = j + NBUF

                @pl.when(jn < CHUNKS)
                def _():
                    pltpu.async_copy(
                        table_sh.at[src_v.at[jn]], gbufs[b], sems[b])
            return 0
        lax.fori_loop(0, CHUNKS // NBUF, group, 0)

        plsc.subcore_barrier()
        pltpu.sync_copy(acc.at[pl.ds(r0, RPT), :],
                        out_hbm.at[cid, pl.ds(r0, RPT), :])

    return k


_deg_kernel = _make_deg_kernel()
_msg16 = _make_msg_kernel(D1)
_msg48 = _make_msg_kernel(D2P)


def _mm1_body(x_ref, w_ref, dp0_ref, dp1_ref, hs_ref, dinv_ref):
    deg = dp0_ref[0, :, 0:1] + dp1_ref[0, :, 0:1] + 1.0
    dinv = lax.rsqrt(deg)
    dinv_ref[...] = dinv
    h = jnp.dot(x_ref[...], w_ref[...], preferred_element_type=jnp.float32)
    hs_ref[...] = h * dinv


def _mm1(x, W1, degp):
    return pl.pallas_call(
        _mm1_body,
        grid=(N // RB,),
        in_specs=[
            pl.BlockSpec((RB, 128), lambda i: (i, 0)),
            pl.BlockSpec((128, D1), lambda i: (0, 0)),
            pl.BlockSpec((1, RB, 16), lambda i: (0, i, 0)),
            pl.BlockSpec((1, RB, 16), lambda i: (1, i, 0)),
        ],
        out_specs=[
            pl.BlockSpec((RB, D1), lambda i: (i, 0)),
            pl.BlockSpec((RB, 1), lambda i: (i, 0)),
        ],
        out_shape=[
            jax.ShapeDtypeStruct((N, D1), jnp.float32),
            jax.ShapeDtypeStruct((N, 1), jnp.float32),
        ],
    )(x, W1, degp, degp)


def _comb1_body(a0_ref, a1_ref, hs_ref, dinv_ref, b1_ref, w2_ref, hs2_ref):
    dinv = dinv_ref[...]
    s = (a0_ref[0] + a1_ref[0] + hs_ref[...]) * dinv + b1_ref[...]
    o1 = jnp.maximum(s, 0.0)
    hs2_ref[...] = jnp.dot(
        o1, w2_ref[...], preferred_element_type=jnp.float32) * dinv


def _comb1(acc1, hs1, dinv, b1r, W2):
    return pl.pallas_call(
        _comb1_body,
        grid=(N // RB,),
        in_specs=[
            pl.BlockSpec((1, RB, D1), lambda i: (0, i, 0)),
            pl.BlockSpec((1, RB, D1), lambda i: (1, i, 0)),
            pl.BlockSpec((RB, D1), lambda i: (i, 0)),
            pl.BlockSpec((RB, 1), lambda i: (i, 0)),
            pl.BlockSpec((1, D1), lambda i: (0, 0)),
            pl.BlockSpec((D1, D2P), lambda i: (0, 0)),
        ],
        out_specs=pl.BlockSpec((RB, D2P), lambda i: (i, 0)),
        out_shape=jax.ShapeDtypeStruct((N, D2P), jnp.float32),
    )(acc1, acc1, hs1, dinv, b1r, W2)


def _final_body(a0_ref, a1_ref, hs_ref, dinv_ref, b2_ref, out_ref):
    z = (a0_ref[0] + a1_ref[0] + hs_ref[...]) * dinv_ref[...] + b2_ref[...]
    m = jnp.max(z, axis=1, keepdims=True)
    lse = jnp.log(jnp.sum(jnp.exp(z - m), axis=1, keepdims=True))
    out_ref[...] = z - m - lse


def _final(acc2, hs2, dinv, b2r):
    return pl.pallas_call(
        _final_body,
        grid=(N // RB,),
        in_specs=[
            pl.BlockSpec((1, RB, D2P), lambda i: (0, i, 0)),
            pl.BlockSpec((1, RB, D2P), lambda i: (1, i, 0)),
            pl.BlockSpec((RB, D2P), lambda i: (i, 0)),
            pl.BlockSpec((RB, 1), lambda i: (i, 0)),
            pl.BlockSpec((1, D2P), lambda i: (0, 0)),
        ],
        out_specs=pl.BlockSpec((RB, D2P), lambda i: (i, 0)),
        out_shape=jax.ShapeDtypeStruct((N, D2P), jnp.float32),
    )(acc2, acc2, hs2, dinv, b2r)


def kernel(x, edge_index, W1, b1, W2, b2):
    er = jnp.asarray(edge_index, jnp.int32).reshape(2, NW, CHUNKS, EC)
    b1r = b1.reshape(1, D1)
    b2r = b2.reshape(1, D2P)
    z16 = jnp.zeros((RPT, D1), jnp.float32)
    z40 = jnp.zeros((RPT, D2P), jnp.float32)

    degp = _deg_kernel(er)
    hs1, dinv = _mm1(x, W1, degp)
    acc1 = _msg16(hs1, er, z16)
    hs2 = _comb1(acc1, hs1, dinv, b1r, W2)
    acc2 = _msg48(hs2, er, z40)
    return _final(acc2, hs2, dinv, b2r)


# speedup vs baseline: 1.0006x; 1.0006x over previous
import functools

import jax
import jax.numpy as jnp
from jax import lax
from jax.experimental import pallas as pl
from jax.experimental.pallas import tpu as pltpu
from jax.experimental.pallas import tpu_sc as plsc

N = 10000
E = 320000
NC, NS = 2, 16
NW = NC * NS
CHUNKS = 80
EC = 125
RPT = N // NS
DW = 8
D1 = 16
D2P = 40
RB = 10000
NBUF = 4


def _make_deg_kernel():
    mesh = plsc.VectorSubcoreMesh(core_axis_name="c", subcore_axis_name="s")

    @functools.partial(
        pl.kernel, mesh=mesh,
        out_type=jax.ShapeDtypeStruct((NC, N, DW), jnp.float32),
        compiler_params=pltpu.CompilerParams(use_tc_tiling_on_sc=False),
        scratch_types=[
            pltpu.VMEM((CHUNKS, EC), jnp.int32),
            pltpu.VMEM((EC, DW), jnp.float32),
            pltpu.VMEM_SHARED((N, DW), jnp.float32),
        ],
    )
    def k(er_hbm, ones_hbm, zeros_hbm, out_hbm, dst_v, obuf, acc):
        cid = lax.axis_index("c")
        sid = lax.axis_index("s")
        wid = sid * NC + cid

        r0 = sid * RPT
        pltpu.sync_copy(zeros_hbm, acc.at[pl.ds(r0, RPT), :])
        pltpu.sync_copy(ones_hbm, obuf)
        pltpu.sync_copy(er_hbm.at[1, wid], dst_v)
        plsc.subcore_barrier()

        def body(j, _):
            pltpu.sync_copy(obuf, acc.at[dst_v.at[j]], add=True)
            return 0
        lax.fori_loop(0, CHUNKS, body, 0)

        plsc.subcore_barrier()
        pltpu.sync_copy(acc.at[pl.ds(r0, RPT), :],
                        out_hbm.at[cid, pl.ds(r0, RPT), :])

    return k


def _make_msg_kernel(D):
    mesh = plsc.VectorSubcoreMesh(core_axis_name="c", subcore_axis_name="s")

    @functools.partial(
        pl.kernel, mesh=mesh,
        out_type=jax.ShapeDtypeStruct((NC, N, D), jnp.float32),
        compiler_params=pltpu.CompilerParams(use_tc_tiling_on_sc=False),
        scratch_types=[
            pltpu.VMEM((CHUNKS, EC), jnp.int32),
            pltpu.VMEM((CHUNKS, EC), jnp.int32),
            [pltpu.VMEM((EC, D), jnp.float32) for _ in range(NBUF)],
            [pltpu.SemaphoreType.DMA for _ in range(NBUF)],
            pltpu.VMEM((RPT, D), jnp.float32),
            pltpu.SemaphoreType.DMA,
            pltpu.VMEM_SHARED((N, D), jnp.float32),
            pltpu.VMEM_SHARED((N, D), jnp.float32),
        ],
    )
    def k(table_hbm, er_hbm, zeros_hbm, out_hbm, src_v, dst_v,
          gbufs, sems, stage_v, sem_s, table_sh, acc):
        cid = lax.axis_index("c")
        sid = lax.axis_index("s")
        wid = sid * NC + cid
        r0 = sid * RPT

        pltpu.async_copy(table_hbm.at[pl.ds(r0, RPT), :], stage_v, sem_s)
        pltpu.sync_copy(zeros_hbm, acc.at[pl.ds(r0, RPT), :])
        pltpu.sync_copy(er_hbm.at[0, wid], src_v)
        pltpu.sync_copy(er_hbm.at[1, wid], dst_v)
        pltpu.make_async_copy(
            table_hbm.at[pl.ds(r0, RPT), :], stage_v, sem_s).wait()
        pltpu.sync_copy(stage_v, table_sh.at[pl.ds(r0, RPT), :])
        plsc.subcore_barrier()

        for b in range(NBUF):
            pltpu.async_copy(table_sh.at[src_v.at[b]], gbufs[b], sems[b])

        def group(g, _):
            base = g * NBUF
            for b in range(NBUF):
                j = base + b
                pltpu.make_async_copy(
                    table_sh.at[src_v.at[j]], gbufs[b], sems[b]).wait()
                pltpu.sync_copy(gbufs[b], acc.at[dst_v.at[j]], add=True)
                jn = j + NBUF

                @pl.when(jn < CHUNKS)
                def _():
                    pltpu.async_copy(
                        table_sh.at[src_v.at[jn]], gbufs[b], sems[b])
            return 0
        lax.fori_loop(0, CHUNKS // NBUF, group, 0)

        plsc.subcore_barrier()
        pltpu.sync_copy(acc.at[pl.ds(r0, RPT), :],
                        out_hbm.at[cid, pl.ds(r0, RPT), :])

    return k


_deg_kernel = _make_deg_kernel()
_msg16 = _make_msg_kernel(D1)
_msg48 = _make_msg_kernel(D2P)


def _mm1_body(x_ref, w_ref, dp0_ref, dp1_ref, hs_ref, dinv_ref):
    deg = dp0_ref[0, :, 0:1] + dp1_ref[0, :, 0:1] + 1.0
    dinv = lax.rsqrt(deg)
    dinv_ref[...] = dinv
    h = jnp.dot(x_ref[...], w_ref[...], preferred_element_type=jnp.float32)
    hs_ref[...] = h * dinv


def _mm1(x, W1, degp):
    return pl.pallas_call(
        _mm1_body,
        grid=(N // RB,),
        in_specs=[
            pl.BlockSpec((RB, 128), lambda i: (i, 0)),
            pl.BlockSpec((128, D1), lambda i: (0, 0)),
            pl.BlockSpec((1, RB, DW), lambda i: (0, i, 0)),
            pl.BlockSpec((1, RB, DW), lambda i: (1, i, 0)),
        ],
        out_specs=[
            pl.BlockSpec((RB, D1), lambda i: (i, 0)),
            pl.BlockSpec((RB, 1), lambda i: (i, 0)),
        ],
        out_shape=[
            jax.ShapeDtypeStruct((N, D1), jnp.float32),
            jax.ShapeDtypeStruct((N, 1), jnp.float32),
        ],
    )(x, W1, degp, degp)


def _comb1_body(a0_ref, a1_ref, hs_ref, dinv_ref, b1_ref, w2_ref, hs2_ref):
    dinv = dinv_ref[...]
    s = (a0_ref[0] + a1_ref[0] + hs_ref[...]) * dinv + b1_ref[...]
    o1 = jnp.maximum(s, 0.0)
    hs2_ref[...] = jnp.dot(
        o1, w2_ref[...], preferred_element_type=jnp.float32) * dinv


def _comb1(acc1, hs1, dinv, b1r, W2):
    return pl.pallas_call(
        _comb1_body,
        grid=(N // RB,),
        in_specs=[
            pl.BlockSpec((1, RB, D1), lambda i: (0, i, 0)),
            pl.BlockSpec((1, RB, D1), lambda i: (1, i, 0)),
            pl.BlockSpec((RB, D1), lambda i: (i, 0)),
            pl.BlockSpec((RB, 1), lambda i: (i, 0)),
            pl.BlockSpec((1, D1), lambda i: (0, 0)),
            pl.BlockSpec((D1, D2P), lambda i: (0, 0)),
        ],
        out_specs=pl.BlockSpec((RB, D2P), lambda i: (i, 0)),
        out_shape=jax.ShapeDtypeStruct((N, D2P), jnp.float32),
    )(acc1, acc1, hs1, dinv, b1r, W2)


def _final_body(a0_ref, a1_ref, hs_ref, dinv_ref, b2_ref, out_ref):
    z = (a0_ref[0] + a1_ref[0] + hs_ref[...]) * dinv_ref[...] + b2_ref[...]
    m = jnp.max(z, axis=1, keepdims=True)
    lse = jnp.log(jnp.sum(jnp.exp(z - m), axis=1, keepdims=True))
    out_ref[...] = z - m - lse


def _final(acc2, hs2, dinv, b2r):
    return pl.pallas_call(
        _final_body,
        grid=(N // RB,),
        in_specs=[
            pl.BlockSpec((1, RB, D2P), lambda i: (0, i, 0)),
            pl.BlockSpec((1, RB, D2P), lambda i: (1, i, 0)),
            pl.BlockSpec((RB, D2P), lambda i: (i, 0)),
            pl.BlockSpec((RB, 1), lambda i: (i, 0)),
            pl.BlockSpec((1, D2P), lambda i: (0, 0)),
        ],
        out_specs=pl.BlockSpec((RB, D2P), lambda i: (i, 0)),
        out_shape=jax.ShapeDtypeStruct((N, D2P), jnp.float32),
    )(acc2, acc2, hs2, dinv, b2r)


def kernel(x, edge_index, W1, b1, W2, b2):
    er = jnp.asarray(edge_index, jnp.int32).reshape(2, NW, CHUNKS, EC)
    b1r = b1.reshape(1, D1)
    b2r = b2.reshape(1, D2P)
    z16 = jnp.zeros((RPT, D1), jnp.float32)
    z40 = jnp.zeros((RPT, D2P), jnp.float32)

    ones8 = jnp.ones((EC, DW), jnp.float32)
    z8 = jnp.zeros((RPT, DW), jnp.float32)
    degp = _deg_kernel(er, ones8, z8)
    hs1, dinv = _mm1(x, W1, degp)
    acc1 = _msg16(hs1, er, z16)
    hs2 = _comb1(acc1, hs1, dinv, b1r, W2)
    acc2 = _msg48(hs2, er, z40)
    return _final(acc2, hs2, dinv, b2r)


# speedup vs baseline: 1.0041x; 1.0035x over previous
import functools

import jax
import jax.numpy as jnp
from jax import lax
from jax.experimental import pallas as pl
from jax.experimental.pallas import tpu as pltpu
from jax.experimental.pallas import tpu_sc as plsc

N = 10000
E = 320000
NC, NS = 2, 16
NW = NC * NS
CHUNKS = 80
EC = 125
RPT = N // NS
DW = 8
D1 = 16
D2P = 40
RB = 2000
NBUF = 5


def _make_deg_kernel():
    mesh = plsc.VectorSubcoreMesh(core_axis_name="c", subcore_axis_name="s")

    @functools.partial(
        pl.kernel, mesh=mesh,
        out_type=jax.ShapeDtypeStruct((NC, N, DW), jnp.float32),
        compiler_params=pltpu.CompilerParams(use_tc_tiling_on_sc=False),
        scratch_types=[
            pltpu.VMEM((CHUNKS, EC), jnp.int32),
            pltpu.VMEM((EC, DW), jnp.float32),
            pltpu.VMEM_SHARED((N, DW), jnp.float32),
        ],
    )
    def k(er_hbm, ones_hbm, zeros_hbm, out_hbm, dst_v, obuf, acc):
        cid = lax.axis_index("c")
        sid = lax.axis_index("s")
        wid = sid * NC + cid

        r0 = sid * RPT
        pltpu.sync_copy(zeros_hbm, acc.at[pl.ds(r0, RPT), :])
        pltpu.sync_copy(ones_hbm, obuf)
        pltpu.sync_copy(er_hbm.at[1, wid], dst_v)
        plsc.subcore_barrier()

        def body(j, _):
            pltpu.sync_copy(obuf, acc.at[dst_v.at[j]], add=True)
            return 0
        lax.fori_loop(0, CHUNKS, body, 0)

        plsc.subcore_barrier()
        pltpu.sync_copy(acc.at[pl.ds(r0, RPT), :],
                        out_hbm.at[cid, pl.ds(r0, RPT), :])

    return k


def _make_msg_kernel(D):
    mesh = plsc.VectorSubcoreMesh(core_axis_name="c", subcore_axis_name="s")

    @functools.partial(
        pl.kernel, mesh=mesh,
        out_type=jax.ShapeDtypeStruct((NC, N, D), jnp.float32),
        compiler_params=pltpu.CompilerParams(use_tc_tiling_on_sc=False),
        scratch_types=[
            pltpu.VMEM((CHUNKS, EC), jnp.int32),
            pltpu.VMEM((CHUNKS, EC), jnp.int32),
            [pltpu.VMEM((EC, D), jnp.float32) for _ in range(NBUF)],
            [pltpu.SemaphoreType.DMA for _ in range(NBUF)],
            pltpu.VMEM((RPT, D), jnp.float32),
            pltpu.SemaphoreType.DMA,
            pltpu.VMEM_SHARED((N, D), jnp.float32),
            pltpu.VMEM_SHARED((N, D), jnp.float32),
        ],
    )
    def k(table_hbm, er_hbm, zeros_hbm, out_hbm, src_v, dst_v,
          gbufs, sems, stage_v, sem_s, table_sh, acc):
        cid = lax.axis_index("c")
        sid = lax.axis_index("s")
        wid = sid * NC + cid
        r0 = sid * RPT

        pltpu.async_copy(table_hbm.at[pl.ds(r0, RPT), :], stage_v, sem_s)
        pltpu.sync_copy(zeros_hbm, acc.at[pl.ds(r0, RPT), :])
        pltpu.sync_copy(er_hbm.at[0, wid], src_v)
        pltpu.sync_copy(er_hbm.at[1, wid], dst_v)
        pltpu.make_async_copy(
            table_hbm.at[pl.ds(r0, RPT), :], stage_v, sem_s).wait()
        pltpu.sync_copy(stage_v, table_sh.at[pl.ds(r0, RPT), :])
        plsc.subcore_barrier()

        for b in range(NBUF):
            pltpu.async_copy(table_sh.at[src_v.at[b]], gbufs[b], sems[b])

        def group(g, _):
            base = g * NBUF
            for b in range(NBUF):
                j = base + b
                pltpu.make_async_copy(
                    table_sh.at[src_v.at[j]], gbufs[b], sems[b]).wait()
                pltpu.sync_copy(gbufs[b], acc.at[dst_v.at[j]], add=True)
                jn = j + NBUF

                @pl.when(jn < CHUNKS)
                def _():
                    pltpu.async_copy(
                        table_sh.at[src_v.at[jn]], gbufs[b], sems[b])
            return 0
        lax.fori_loop(0, CHUNKS // NBUF, group, 0)

        plsc.subcore_barrier()
        pltpu.sync_copy(acc.at[pl.ds(r0, RPT), :],
                        out_hbm.at[cid, pl.ds(r0, RPT), :])

    return k


_deg_kernel = _make_deg_kernel()
_msg16 = _make_msg_kernel(D1)
_msg48 = _make_msg_kernel(D2P)


def _mm1_body(x_ref, w_ref, dp0_ref, dp1_ref, hs_ref, dinv_ref):
    deg = dp0_ref[0, :, 0:1] + dp1_ref[0, :, 0:1] + 1.0
    dinv = lax.rsqrt(deg)
    dinv_ref[...] = dinv
    h = jnp.dot(x_ref[...], w_ref[...], preferred_element_type=jnp.float32)
    hs_ref[...] = h * dinv


def _mm1(x, W1, degp):
    return pl.pallas_call(
        _mm1_body,
        grid=(N // RB,),
        in_specs=[
            pl.BlockSpec((RB, 128), lambda i: (i, 0)),
            pl.BlockSpec((128, D1), lambda i: (0, 0)),
            pl.BlockSpec((1, RB, DW), lambda i: (0, i, 0)),
            pl.BlockSpec((1, RB, DW), lambda i: (1, i, 0)),
        ],
        out_specs=[
            pl.BlockSpec((RB, D1), lambda i: (i, 0)),
            pl.BlockSpec((RB, 1), lambda i: (i, 0)),
        ],
        out_shape=[
            jax.ShapeDtypeStruct((N, D1), jnp.float32),
            jax.ShapeDtypeStruct((N, 1), jnp.float32),
        ],
    )(x, W1, degp, degp)


def _comb1_body(a0_ref, a1_ref, hs_ref, dinv_ref, b1_ref, w2_ref, hs2_ref):
    dinv = dinv_ref[...]
    s = (a0_ref[0] + a1_ref[0] + hs_ref[...]) * dinv + b1_ref[...]
    o1 = jnp.maximum(s, 0.0)
    hs2_ref[...] = jnp.dot(
        o1, w2_ref[...], preferred_element_type=jnp.float32) * dinv


def _comb1(acc1, hs1, dinv, b1r, W2):
    return pl.pallas_call(
        _comb1_body,
        grid=(N // RB,),
        in_specs=[
            pl.BlockSpec((1, RB, D1), lambda i: (0, i, 0)),
            pl.BlockSpec((1, RB, D1), lambda i: (1, i, 0)),
            pl.BlockSpec((RB, D1), lambda i: (i, 0)),
            pl.BlockSpec((RB, 1), lambda i: (i, 0)),
            pl.BlockSpec((1, D1), lambda i: (0, 0)),
            pl.BlockSpec((D1, D2P), lambda i: (0, 0)),
        ],
        out_specs=pl.BlockSpec((RB, D2P), lambda i: (i, 0)),
        out_shape=jax.ShapeDtypeStruct((N, D2P), jnp.float32),
    )(acc1, acc1, hs1, dinv, b1r, W2)


def _final_body(a0_ref, a1_ref, hs_ref, dinv_ref, b2_ref, out_ref):
    z = (a0_ref[0] + a1_ref[0] + hs_ref[...]) * dinv_ref[...] + b2_ref[...]
    m = jnp.max(z, axis=1, keepdims=True)
    lse = jnp.log(jnp.sum(jnp.exp(z - m), axis=1, keepdims=True))
    out_ref[...] = z - m - lse


def _final(acc2, hs2, dinv, b2r):
    return pl.pallas_call(
        _final_body,
        grid=(N // RB,),
        in_specs=[
            pl.BlockSpec((1, RB, D2P), lambda i: (0, i, 0)),
            pl.BlockSpec((1, RB, D2P), lambda i: (1, i, 0)),
            pl.BlockSpec((RB, D2P), lambda i: (i, 0)),
            pl.BlockSpec((RB, 1), lambda i: (i, 0)),
            pl.BlockSpec((1, D2P), lambda i: (0, 0)),
        ],
        out_specs=pl.BlockSpec((RB, D2P), lambda i: (i, 0)),
        out_shape=jax.ShapeDtypeStruct((N, D2P), jnp.float32),
    )(acc2, acc2, hs2, dinv, b2r)


def kernel(x, edge_index, W1, b1, W2, b2):
    er = jnp.asarray(edge_index, jnp.int32).reshape(2, NW, CHUNKS, EC)
    b1r = b1.reshape(1, D1)
    b2r = b2.reshape(1, D2P)
    z16 = jnp.zeros((RPT, D1), jnp.float32)
    z40 = jnp.zeros((RPT, D2P), jnp.float32)

    ones8 = jnp.ones((EC, DW), jnp.float32)
    z8 = jnp.zeros((RPT, DW), jnp.float32)
    degp = _deg_kernel(er, ones8, z8)
    hs1, dinv = _mm1(x, W1, degp)
    acc1 = _msg16(hs1, er, z16)
    hs2 = _comb1(acc1, hs1, dinv, b1r, W2)
    acc2 = _msg48(hs2, er, z40)
    return _final(acc2, hs2, dinv, b2r)
